# baseline (speedup 1.0000x reference)
import functools

import jax
import jax.numpy as jnp
import numpy as np
from jax.experimental import pallas as pl
from jax.experimental.pallas import tpu as pltpu

_B = 8
_N = 16384
_DIM = 128
_NDIR = 13
_CG = 32
_CO = 32


def _tree_dims():
    d = [8]
    f = 8
    for _ in range(int(np.log2(_N))):
        f = min(f * 2, _DIM)
        d.append(f)
    return d


def _layer_meta():
    dims = _tree_dims()
    meta = []
    n = _N
    off = 0
    for l in range(len(dims) - 1):
        n2 = n // 2
        d = dims[l]
        d2 = dims[l + 1]
        packed = d < _DIM
        meta.append(dict(l=l, off=off, n=n, n2=n2, d=d, d2=d2, packed=packed))
        off += n2
        n = n2
    return dims, meta


def _prelu(x, a):
    return jnp.where(x >= 0, x, a * x)


def _encoder_body(meta, pts_ref, alphas_ref, leafW_ref, leafb_ref, *rest):
    nl = len(meta)
    r_refs = rest[:nl]
    dm_refs = rest[nl:2 * nl]
    w_refs = rest[2 * nl:3 * nl]
    b_refs = rest[3 * nl:4 * nl]
    out_ref = rest[4 * nl]
    a_ref = rest[4 * nl + 1]
    b2_ref = rest[4 * nl + 2]

    def leaf_chunk(i, _):
        x = pts_ref[pl.ds(i * _CG, _CG)]
        x2 = x.reshape(_CG * _B, _DIM)
        y = jnp.dot(x2, leafW_ref[...], preferred_element_type=jnp.float32)
        y = y + leafb_ref[...]
        y = _prelu(y, alphas_ref[0])
        a_ref[pl.ds(i * _CG, _CG)] = y.reshape(_CG, _B, _DIM)
        return 0

    jax.lax.fori_loop(0, 1024 // _CG, leaf_chunk, 0)

    bufs = [a_ref, b2_ref]
    for m in meta:
        l = m["l"]
        src = bufs[l % 2]
        dst = bufs[(l + 1) % 2]
        r_ref = r_refs[l]
        dm_ref = dm_refs[l]
        w_ref = w_refs[l]
        bb_ref = b_refs[l]
        alpha_idx = l + 1

        if m["packed"]:
            d = m["d"]
            groups = m["n"] * d // _DIM
            cg = min(_CG, groups)
            lane = jax.lax.broadcasted_iota(jnp.int32, (1, 1, _DIM), 2)
            is_left = (lane % (2 * d)) < d

            def pk_chunk(i, _, d=d, cg=cg, r_ref=r_ref, dm_ref=dm_ref,
                         w_ref=w_ref, bb_ref=bb_ref, src=src, dst=dst,
                         alpha_idx=alpha_idx, is_left=is_left):
                x = src[pl.ds(i * cg, cg)]
                r = r_ref[pl.ds(i * cg, cg)]
                dm = dm_ref[pl.ds(i * cg, cg)]
                rollm = jnp.concatenate([x[..., d:], x[..., :d]], axis=-1)
                rollp = jnp.concatenate([x[..., -d:], x[..., :-d]], axis=-1)
                partner = jnp.where(is_left, rollm, rollp)
                cat = x + r * (partner - x)
                x2 = cat.reshape(cg * _B, _DIM)
                acc = jnp.zeros((cg, _B, _DIM), jnp.float32)
                for p in range((_NDIR + 1) // 2):
                    pe2 = jnp.dot(x2, w_ref[p],
                                  preferred_element_type=jnp.float32)
                    pe2 = pe2.reshape(cg, _B, 2 * _DIM)
                    for h in range(2):
                        e = 2 * p + h
                        if e >= _NDIR:
                            continue
                        pe = pe2[:, :, h * _DIM:(h + 1) * _DIM] + bb_ref[e]
                        me = (dm == e).astype(jnp.float32)
                        acc = acc + me * pe
                acc = _prelu(acc, alphas_ref[alpha_idx])
                dst[pl.ds(i * cg, cg)] = acc
                return 0

            jax.lax.fori_loop(0, groups // cg, pk_chunk, 0)
        else:
            n2 = m["n2"]
            co = min(_CO, n2)

            def up_chunk(i, _, co=co, r_ref=r_ref, dm_ref=dm_ref,
                         w_ref=w_ref, bb_ref=bb_ref, src=src, dst=dst,
                         alpha_idx=alpha_idx):
                x = src[pl.ds(i * 2 * co, 2 * co)]
                x4 = x.reshape(co, 2, _B, _DIM)
                lch = x4[:, 0]
                rch = x4[:, 1]
                r = r_ref[pl.ds(i * co, co)]
                dm = dm_ref[pl.ds(i * co, co)]
                sel_l = lch + r * (rch - lch)
                sel_r = rch + r * (lch - rch)
                cat = jnp.concatenate([sel_l, sel_r], axis=-1)
                x2 = cat.reshape(co * _B, 2 * _DIM)
                acc = jnp.zeros((co, _B, _DIM), jnp.float32)
                for p in range((_NDIR + 1) // 2):
                    pe2 = jnp.dot(x2, w_ref[p],
                                  preferred_element_type=jnp.float32)
                    pe2 = pe2.reshape(co, _B, 2 * _DIM)
                    for h in range(2):
                        e = 2 * p + h
                        if e >= _NDIR:
                            continue
                        pe = pe2[:, :, h * _DIM:(h + 1) * _DIM] + bb_ref[e]
                        me = (dm == e).astype(jnp.float32)
                        acc = acc + me * pe
                acc = _prelu(acc, alphas_ref[alpha_idx])
                dst[pl.ds(i * co, co)] = acc
                return 0

            jax.lax.fori_loop(0, n2 // co, up_chunk, 0)

    final = bufs[len(meta) % 2]
    out_ref[...] = final[0]


def kernel(points, vec, dmap, drev, leaf_W, leaf_b, merge_Ws, merge_bs, alphas):
    dims, meta = _layer_meta()

    pts = jnp.transpose(points, (1, 0, 2))
    pts = jnp.pad(pts, ((0, 0), (0, 0), (0, 5)))
    pts = pts.reshape(_N // 16, 16, _B, 8)
    pts = jnp.transpose(pts, (0, 2, 1, 3)).reshape(_N // 16, _B, _DIM)

    lWp = jnp.pad(leaf_W, ((0, 5), (0, 0)))
    leafW = jnp.zeros((_DIM, _DIM), jnp.float32)
    for k in range(16):
        leafW = jax.lax.dynamic_update_slice(leafW, lWp, (k * 8, k * 8))
    leafb = jnp.tile(leaf_b, 16).reshape(1, _DIM)

    rv = jnp.zeros(vec.shape, jnp.float32)
    dmv = jnp.zeros(vec.shape, jnp.int32)
    drev_f = drev.astype(jnp.float32)
    for j in range(_NDIR):
        mj = vec == j
        rv = rv + jnp.where(mj, drev_f[j], 0.0)
        dmv = dmv + jnp.where(mj, dmap[j], 0)

    r_in = []
    dm_in = []
    w_in = []
    b_in = []
    for m in meta:
        l = m["l"]
        n2, d2 = m["n2"], m["d2"]
        gshape = (n2 * d2 // _DIM, 1, _DIM)
        rl = jax.lax.dynamic_slice(rv, (m["off"],), (n2,))
        dml = jax.lax.dynamic_slice(dmv, (m["off"],), (n2,))
        r_in.append(jnp.broadcast_to(
            rl.reshape(n2, 1), (n2, d2)).reshape(gshape))
        dm_in.append(jnp.broadcast_to(
            dml.reshape(n2, 1), (n2, d2)).reshape(gshape))
        W = merge_Ws[l]
        bb = merge_bs[l]
        if m["packed"]:
            g2 = _DIM // d2
            bd = jnp.zeros((_NDIR + 1, _DIM, _DIM), jnp.float32)
            for k in range(g2):
                bd = jax.lax.dynamic_update_slice(bd, W, (0, k * d2, k * d2))
            w_in.append(jnp.concatenate([bd[0::2], bd[1::2]], axis=-1))
            b_in.append(jnp.tile(bb, (1, g2)).reshape(_NDIR, 1, _DIM))
        else:
            wp = jnp.concatenate(
                [W, jnp.zeros((1, 2 * _DIM, _DIM), jnp.float32)], axis=0)
            w_in.append(jnp.concatenate([wp[0::2], wp[1::2]], axis=-1))
            b_in.append(bb.reshape(_NDIR, 1, _DIM))

    smem = pl.BlockSpec(memory_space=pltpu.SMEM)
    vmem = pl.BlockSpec(memory_space=pltpu.VMEM)
    nl = len(meta)
    body = functools.partial(_encoder_body, meta)
    return pl.pallas_call(
        body,
        out_shape=jax.ShapeDtypeStruct((_B, _DIM), jnp.float32),
        in_specs=[vmem, smem, vmem, vmem] + [vmem] * (4 * nl),
        out_specs=vmem,
        scratch_shapes=[
            pltpu.VMEM((1024, _B, _DIM), jnp.float32),
            pltpu.VMEM((1024, _B, _DIM), jnp.float32),
        ],
        compiler_params=pltpu.CompilerParams(
            vmem_limit_bytes=100 * 1024 * 1024),
    )(pts, alphas, leafW, leafb, *r_in, *dm_in, *w_in, *b_in)


# speedup vs baseline: 1.0531x; 1.0531x over previous
import functools

import jax
import jax.numpy as jnp
import numpy as np
from jax.experimental import pallas as pl
from jax.experimental.pallas import tpu as pltpu

_B = 8
_N = 16384
_DIM = 128
_NDIR = 13
_CG = 32
_CO = 32


def _tree_dims():
    d = [8]
    f = 8
    for _ in range(int(np.log2(_N))):
        f = min(f * 2, _DIM)
        d.append(f)
    return d


def _layer_meta():
    dims = _tree_dims()
    meta = []
    n = _N
    off = 0
    for l in range(len(dims) - 1):
        n2 = n // 2
        d = dims[l]
        d2 = dims[l + 1]
        packed = d < _DIM
        meta.append(dict(l=l, off=off, n=n, n2=n2, d=d, d2=d2, packed=packed))
        off += n2
        n = n2
    return dims, meta


def _prelu(x, a):
    return jnp.where(x >= 0, x, a * x)


def _encoder_body(meta, pts_ref, alphas_ref, leafW_ref, leafb_ref, *rest):
    nl = len(meta)
    r_refs = rest[:nl]
    dm_refs = rest[nl:2 * nl]
    w_refs = rest[2 * nl:3 * nl]
    b_refs = rest[3 * nl:4 * nl]
    out_ref = rest[4 * nl]
    a_ref = rest[4 * nl + 1]
    b2_ref = rest[4 * nl + 2]

    def leaf_chunk(i, _):
        x = pts_ref[pl.ds(i * _CG, _CG)]
        x2 = x.reshape(_CG * _B, _DIM)
        y = jnp.dot(x2, leafW_ref[...], preferred_element_type=jnp.float32)
        y = y + leafb_ref[...]
        y = _prelu(y, alphas_ref[0])
        a_ref[pl.ds(i * _CG, _CG)] = y.reshape(_CG, _B, _DIM)
        return 0

    jax.lax.fori_loop(0, 1024 // _CG, leaf_chunk, 0)

    bufs = [a_ref, b2_ref]
    for m in meta:
        l = m["l"]
        src = bufs[l % 2]
        dst = bufs[(l + 1) % 2]
        r_ref = r_refs[l]
        dm_ref = dm_refs[l]
        w_ref = w_refs[l]
        bb_ref = b_refs[l]
        alpha_idx = l + 1

        if m["packed"]:
            d = m["d"]
            groups = m["n"] * d // _DIM
            cg = min(_CG, groups)
            lane = jax.lax.broadcasted_iota(jnp.int32, (1, 1, _DIM), 2)
            is_left = (lane % (2 * d)) < d

            def pk_chunk(i, _, d=d, cg=cg, r_ref=r_ref, dm_ref=dm_ref,
                         w_ref=w_ref, bb_ref=bb_ref, src=src, dst=dst,
                         alpha_idx=alpha_idx, is_left=is_left):
                x = src[pl.ds(i * cg, cg)]
                r = r_ref[pl.ds(i * cg, cg)]
                dm = dm_ref[pl.ds(i * cg, cg)]
                rollm = jnp.concatenate([x[..., d:], x[..., :d]], axis=-1)
                rollp = jnp.concatenate([x[..., -d:], x[..., :-d]], axis=-1)
                partner = jnp.where(is_left, rollm, rollp)
                cat = x + r * (partner - x)
                x2 = cat.reshape(cg * _B, _DIM)
                acc = jnp.zeros((cg, _B, _DIM), jnp.float32)
                for e in range(_NDIR):
                    pe = jnp.dot(x2, w_ref[e],
                                 preferred_element_type=jnp.float32)
                    pe = pe.reshape(cg, _B, _DIM) + bb_ref[e]
                    me = (dm == e).astype(jnp.float32)
                    acc = acc + me * pe
                acc = _prelu(acc, alphas_ref[alpha_idx])
                dst[pl.ds(i * cg, cg)] = acc
                return 0

            jax.lax.fori_loop(0, groups // cg, pk_chunk, 0)
        else:
            n2 = m["n2"]
            co = min(_CO, n2)

            def up_chunk(i, _, co=co, r_ref=r_ref, dm_ref=dm_ref,
                         w_ref=w_ref, bb_ref=bb_ref, src=src, dst=dst,
                         alpha_idx=alpha_idx):
                x = src[pl.ds(i * 2 * co, 2 * co)]
                x4 = x.reshape(co, 2, _B, _DIM)
                lch = x4[:, 0]
                rch = x4[:, 1]
                r = r_ref[pl.ds(i * co, co)]
                dm = dm_ref[pl.ds(i * co, co)]
                sel_l = lch + r * (rch - lch)
                sel_r = rch + r * (lch - rch)
                cat = jnp.concatenate([sel_l, sel_r], axis=-1)
                x2 = cat.reshape(co * _B, 2 * _DIM)
                acc = jnp.zeros((co, _B, _DIM), jnp.float32)
                for e in range(_NDIR):
                    pe = jnp.dot(x2, w_ref[e],
                                 preferred_element_type=jnp.float32)
                    pe = pe.reshape(co, _B, _DIM) + bb_ref[e]
                    me = (dm == e).astype(jnp.float32)
                    acc = acc + me * pe
                acc = _prelu(acc, alphas_ref[alpha_idx])
                dst[pl.ds(i * co, co)] = acc
                return 0

            jax.lax.fori_loop(0, n2 // co, up_chunk, 0)

    final = bufs[len(meta) % 2]
    out_ref[...] = final[0]


def kernel(points, vec, dmap, drev, leaf_W, leaf_b, merge_Ws, merge_bs, alphas):
    dims, meta = _layer_meta()

    pts = jnp.transpose(points, (1, 0, 2))
    pts = jnp.pad(pts, ((0, 0), (0, 0), (0, 5)))
    pts = pts.reshape(_N // 16, 16, _B, 8)
    pts = jnp.transpose(pts, (0, 2, 1, 3)).reshape(_N // 16, _B, _DIM)

    lWp = jnp.pad(leaf_W, ((0, 5), (0, 0)))
    leafW = jnp.zeros((_DIM, _DIM), jnp.float32)
    for k in range(16):
        leafW = jax.lax.dynamic_update_slice(leafW, lWp, (k * 8, k * 8))
    leafb = jnp.tile(leaf_b, 16).reshape(1, _DIM)

    rv = jnp.zeros(vec.shape, jnp.float32)
    dmv = jnp.zeros(vec.shape, jnp.int32)
    drev_f = drev.astype(jnp.float32)
    for j in range(_NDIR):
        mj = vec == j
        rv = rv + jnp.where(mj, drev_f[j], 0.0)
        dmv = dmv + jnp.where(mj, dmap[j], 0)

    r_in = []
    dm_in = []
    w_in = []
    b_in = []
    for m in meta:
        l = m["l"]
        n2, d2 = m["n2"], m["d2"]
        gshape = (n2 * d2 // _DIM, 1, _DIM)
        rl = jax.lax.dynamic_slice(rv, (m["off"],), (n2,))
        dml = jax.lax.dynamic_slice(dmv, (m["off"],), (n2,))
        r_in.append(jnp.broadcast_to(
            rl.reshape(n2, 1), (n2, d2)).reshape(gshape))
        dm_in.append(jnp.broadcast_to(
            dml.reshape(n2, 1), (n2, d2)).reshape(gshape))
        W = merge_Ws[l]
        bb = merge_bs[l]
        if m["packed"]:
            g2 = _DIM // d2
            bd = jnp.zeros((_NDIR, _DIM, _DIM), jnp.float32)
            for k in range(g2):
                bd = jax.lax.dynamic_update_slice(bd, W, (0, k * d2, k * d2))
            w_in.append(bd)
            b_in.append(jnp.tile(bb, (1, g2)).reshape(_NDIR, 1, _DIM))
        else:
            w_in.append(W)
            b_in.append(bb.reshape(_NDIR, 1, _DIM))

    smem = pl.BlockSpec(memory_space=pltpu.SMEM)
    vmem = pl.BlockSpec(memory_space=pltpu.VMEM)
    nl = len(meta)
    body = functools.partial(_encoder_body, meta)
    return pl.pallas_call(
        body,
        out_shape=jax.ShapeDtypeStruct((_B, _DIM), jnp.float32),
        in_specs=[vmem, smem, vmem, vmem] + [vmem] * (4 * nl),
        out_specs=vmem,
        scratch_shapes=[
            pltpu.VMEM((1024, _B, _DIM), jnp.float32),
            pltpu.VMEM((1024, _B, _DIM), jnp.float32),
        ],
        compiler_params=pltpu.CompilerParams(
            vmem_limit_bytes=100 * 1024 * 1024),
    )(pts, alphas, leafW, leafb, *r_in, *dm_in, *w_in, *b_in)


# speedup vs baseline: 1.1025x; 1.0469x over previous
import functools

import jax
import jax.numpy as jnp
import numpy as np
from jax.experimental import pallas as pl
from jax.experimental.pallas import tpu as pltpu

_B = 8
_N = 16384
_DIM = 128
_NDIR = 13
_CG = 32
_CO = 32


def _tree_dims():
    d = [8]
    f = 8
    for _ in range(int(np.log2(_N))):
        f = min(f * 2, _DIM)
        d.append(f)
    return d


def _layer_meta():
    dims = _tree_dims()
    meta = []
    n = _N
    off = 0
    for l in range(len(dims) - 1):
        n2 = n // 2
        d = dims[l]
        d2 = dims[l + 1]
        packed = d < _DIM
        meta.append(dict(l=l, off=off, n=n, n2=n2, d=d, d2=d2, packed=packed))
        off += n2
        n = n2
    return dims, meta


def _prelu(x, a):
    return jnp.where(x >= 0, x, a * x)


def _route_masks(ve, dmap_ref, drev_ref, rows):
    r = jnp.zeros((rows, 1, _DIM), jnp.float32)
    dm = jnp.zeros((rows, 1, _DIM), jnp.int32)
    for j in range(_NDIR):
        mj = ve == j
        r = r + jnp.where(mj, drev_ref[j], 0.0)
        dm = dm + jnp.where(mj, dmap_ref[j], 0)
    return r, dm


def _encoder_body(meta, pts_ref, dmap_ref, drev_ref, alphas_ref, leafW_ref,
                  leafb_ref, *rest):
    nl = len(meta)
    ve_refs = rest[:nl]
    w_refs = rest[nl:2 * nl]
    b_refs = rest[2 * nl:3 * nl]
    out_ref = rest[3 * nl]
    a_ref = rest[3 * nl + 1]
    b2_ref = rest[3 * nl + 2]

    def leaf_chunk(i, _):
        x = pts_ref[pl.ds(i * _CG, _CG)]
        x2 = x.reshape(_CG * _B, _DIM)
        y = jnp.dot(x2, leafW_ref[...], preferred_element_type=jnp.float32)
        y = y + leafb_ref[...]
        y = _prelu(y, alphas_ref[0])
        a_ref[pl.ds(i * _CG, _CG)] = y.reshape(_CG, _B, _DIM)
        return 0

    jax.lax.fori_loop(0, 1024 // _CG, leaf_chunk, 0)

    bufs = [a_ref, b2_ref]
    for m in meta:
        l = m["l"]
        src = bufs[l % 2]
        dst = bufs[(l + 1) % 2]
        ve_ref = ve_refs[l]
        w_ref = w_refs[l]
        bb_ref = b_refs[l]
        alpha_idx = l + 1

        if m["packed"]:
            d = m["d"]
            groups = m["n"] * d // _DIM
            cg = min(_CG, groups)
            lane = jax.lax.broadcasted_iota(jnp.int32, (1, 1, _DIM), 2)
            is_left = (lane % (2 * d)) < d

            def pk_chunk(i, _, d=d, cg=cg, ve_ref=ve_ref, w_ref=w_ref,
                         bb_ref=bb_ref, src=src, dst=dst,
                         alpha_idx=alpha_idx, is_left=is_left):
                x = src[pl.ds(i * cg, cg)]
                ve = ve_ref[pl.ds(i * cg, cg)]
                r, dm = _route_masks(ve, dmap_ref, drev_ref, cg)
                rollm = jnp.concatenate([x[..., d:], x[..., :d]], axis=-1)
                rollp = jnp.concatenate([x[..., -d:], x[..., :-d]], axis=-1)
                partner = jnp.where(is_left, rollm, rollp)
                cat = x + r * (partner - x)
                x2 = cat.reshape(cg * _B, _DIM)
                acc = jnp.zeros((cg, _B, _DIM), jnp.float32)
                for e in range(_NDIR):
                    pe = jnp.dot(x2, w_ref[e],
                                 preferred_element_type=jnp.float32)
                    pe = pe.reshape(cg, _B, _DIM) + bb_ref[e]
                    me = (dm == e).astype(jnp.float32)
                    acc = acc + me * pe
                acc = _prelu(acc, alphas_ref[alpha_idx])
                dst[pl.ds(i * cg, cg)] = acc
                return 0

            jax.lax.fori_loop(0, groups // cg, pk_chunk, 0)
        else:
            n2 = m["n2"]
            co = min(_CO, n2)

            def up_chunk(i, _, co=co, ve_ref=ve_ref, w_ref=w_ref,
                         bb_ref=bb_ref, src=src, dst=dst,
                         alpha_idx=alpha_idx):
                x = src[pl.ds(i * 2 * co, 2 * co)]
                x4 = x.reshape(co, 2, _B, _DIM)
                lch = x4[:, 0]
                rch = x4[:, 1]
                ve = ve_ref[pl.ds(i * co, co)]
                r, dm = _route_masks(ve, dmap_ref, drev_ref, co)
                sel_l = lch + r * (rch - lch)
                sel_r = rch + r * (lch - rch)
                cat = jnp.concatenate([sel_l, sel_r], axis=-1)
                x2 = cat.reshape(co * _B, 2 * _DIM)
                acc = jnp.zeros((co, _B, _DIM), jnp.float32)
                for e in range(_NDIR):
                    pe = jnp.dot(x2, w_ref[e],
                                 preferred_element_type=jnp.float32)
                    pe = pe.reshape(co, _B, _DIM) + bb_ref[e]
                    me = (dm == e).astype(jnp.float32)
                    acc = acc + me * pe
                acc = _prelu(acc, alphas_ref[alpha_idx])
                dst[pl.ds(i * co, co)] = acc
                return 0

            jax.lax.fori_loop(0, n2 // co, up_chunk, 0)

    final = bufs[len(meta) % 2]
    out_ref[...] = final[0]


def kernel(points, vec, dmap, drev, leaf_W, leaf_b, merge_Ws, merge_bs, alphas):
    dims, meta = _layer_meta()

    pts = jnp.transpose(points, (1, 0, 2))
    pts = jnp.pad(pts, ((0, 0), (0, 0), (0, 5)))
    pts = pts.reshape(_N // 16, 16, _B, 8)
    pts = jnp.transpose(pts, (0, 2, 1, 3)).reshape(_N // 16, _B, _DIM)

    lWp = jnp.pad(leaf_W, ((0, 5), (0, 0)))
    leafW = jnp.zeros((_DIM, _DIM), jnp.float32)
    for k in range(16):
        leafW = jax.lax.dynamic_update_slice(leafW, lWp, (k * 8, k * 8))
    leafb = jnp.tile(leaf_b, 16).reshape(1, _DIM)

    ve_in = []
    w_in = []
    b_in = []
    for m in meta:
        l = m["l"]
        n2, d2 = m["n2"], m["d2"]
        v = jax.lax.dynamic_slice(vec, (m["off"],), (n2,))
        ve = jnp.broadcast_to(v.reshape(n2, 1), (n2, d2))
        ve_in.append(ve.reshape(n2 * d2 // _DIM, 1, _DIM))
        W = merge_Ws[l]
        bb = merge_bs[l]
        if m["packed"]:
            g2 = _DIM // d2
            bd = jnp.zeros((_NDIR, _DIM, _DIM), jnp.float32)
            for k in range(g2):
                bd = jax.lax.dynamic_update_slice(bd, W, (0, k * d2, k * d2))
            w_in.append(bd)
            b_in.append(jnp.tile(bb, (1, g2)).reshape(_NDIR, 1, _DIM))
        else:
            w_in.append(W)
            b_in.append(bb.reshape(_NDIR, 1, _DIM))

    smem = pl.BlockSpec(memory_space=pltpu.SMEM)
    vmem = pl.BlockSpec(memory_space=pltpu.VMEM)
    nl = len(meta)
    body = functools.partial(_encoder_body, meta)
    return pl.pallas_call(
        body,
        out_shape=jax.ShapeDtypeStruct((_B, _DIM), jnp.float32),
        in_specs=[vmem, smem, smem, smem, vmem, vmem] + [vmem] * (3 * nl),
        out_specs=vmem,
        scratch_shapes=[
            pltpu.VMEM((1024, _B, _DIM), jnp.float32),
            pltpu.VMEM((1024, _B, _DIM), jnp.float32),
        ],
        compiler_params=pltpu.CompilerParams(
            vmem_limit_bytes=100 * 1024 * 1024),
    )(pts, dmap, drev.astype(jnp.float32), alphas, leafW, leafb,
      *ve_in, *w_in, *b_in)


# speedup vs baseline: 1.1401x; 1.0341x over previous
import functools

import jax
import jax.numpy as jnp
import numpy as np
from jax.experimental import pallas as pl
from jax.experimental.pallas import tpu as pltpu

_B = 8
_N = 16384
_DIM = 128
_NDIR = 13
_CG = 64
_CO = 64


def _tree_dims():
    d = [8]
    f = 8
    for _ in range(int(np.log2(_N))):
        f = min(f * 2, _DIM)
        d.append(f)
    return d


def _layer_meta():
    dims = _tree_dims()
    meta = []
    n = _N
    off = 0
    for l in range(len(dims) - 1):
        n2 = n // 2
        d = dims[l]
        d2 = dims[l + 1]
        packed = d < _DIM
        meta.append(dict(l=l, off=off, n=n, n2=n2, d=d, d2=d2, packed=packed))
        off += n2
        n = n2
    return dims, meta


def _prelu(x, a):
    return jnp.where(x >= 0, x, a * x)


def _route_masks(ve, dmap_ref, drev_ref, rows):
    r = jnp.zeros((rows, 1, _DIM), jnp.float32)
    dm = jnp.zeros((rows, 1, _DIM), jnp.int32)
    for j in range(_NDIR):
        mj = ve == j
        r = r + jnp.where(mj, drev_ref[j], 0.0)
        dm = dm + jnp.where(mj, dmap_ref[j], 0)
    return r, dm


def _encoder_body(meta, pts_ref, dmap_ref, drev_ref, alphas_ref, leafW_ref,
                  leafb_ref, *rest):
    nl = len(meta)
    ve_refs = rest[:nl]
    w_refs = rest[nl:2 * nl]
    b_refs = rest[2 * nl:3 * nl]
    out_ref = rest[3 * nl]
    a_ref = rest[3 * nl + 1]
    b2_ref = rest[3 * nl + 2]

    def leaf_chunk(i, _):
        x = pts_ref[pl.ds(i * _CG, _CG)]
        x2 = x.reshape(_CG * _B, _DIM)
        y = jnp.dot(x2, leafW_ref[...], preferred_element_type=jnp.float32)
        y = y + leafb_ref[...]
        y = _prelu(y, alphas_ref[0])
        a_ref[pl.ds(i * _CG, _CG)] = y.reshape(_CG, _B, _DIM)
        return 0

    jax.lax.fori_loop(0, 1024 // _CG, leaf_chunk, 0)

    bufs = [a_ref, b2_ref]
    for m in meta:
        l = m["l"]
        src = bufs[l % 2]
        dst = bufs[(l + 1) % 2]
        ve_ref = ve_refs[l]
        w_ref = w_refs[l]
        bb_ref = b_refs[l]
        alpha_idx = l + 1

        if m["packed"]:
            d = m["d"]
            groups = m["n"] * d // _DIM
            cg = min(_CG, groups)
            lane = jax.lax.broadcasted_iota(jnp.int32, (1, 1, _DIM), 2)
            is_left = (lane % (2 * d)) < d

            def pk_chunk(i, _, d=d, cg=cg, ve_ref=ve_ref, w_ref=w_ref,
                         bb_ref=bb_ref, src=src, dst=dst,
                         alpha_idx=alpha_idx, is_left=is_left):
                x = src[pl.ds(i * cg, cg)]
                ve = ve_ref[pl.ds(i * cg, cg)]
                r, dm = _route_masks(ve, dmap_ref, drev_ref, cg)
                rollm = jnp.concatenate([x[..., d:], x[..., :d]], axis=-1)
                rollp = jnp.concatenate([x[..., -d:], x[..., :-d]], axis=-1)
                partner = jnp.where(is_left, rollm, rollp)
                cat = x + r * (partner - x)
                x2 = cat.reshape(cg * _B, _DIM)
                acc = jnp.zeros((cg, _B, _DIM), jnp.float32)
                for e in range(_NDIR):
                    pe = jnp.dot(x2, w_ref[e],
                                 preferred_element_type=jnp.float32)
                    pe = pe.reshape(cg, _B, _DIM) + bb_ref[e]
                    me = (dm == e).astype(jnp.float32)
                    acc = acc + me * pe
                acc = _prelu(acc, alphas_ref[alpha_idx])
                dst[pl.ds(i * cg, cg)] = acc
                return 0

            jax.lax.fori_loop(0, groups // cg, pk_chunk, 0)
        else:
            n2 = m["n2"]
            co = min(_CO, n2)

            def up_chunk(i, _, co=co, ve_ref=ve_ref, w_ref=w_ref,
                         bb_ref=bb_ref, src=src, dst=dst,
                         alpha_idx=alpha_idx):
                x = src[pl.ds(i * 2 * co, 2 * co)]
                x4 = x.reshape(co, 2, _B, _DIM)
                lch = x4[:, 0]
                rch = x4[:, 1]
                ve = ve_ref[pl.ds(i * co, co)]
                r, dm = _route_masks(ve, dmap_ref, drev_ref, co)
                sel_l = lch + r * (rch - lch)
                sel_r = rch + r * (lch - rch)
                cat = jnp.concatenate([sel_l, sel_r], axis=-1)
                x2 = cat.reshape(co * _B, 2 * _DIM)
                acc = jnp.zeros((co, _B, _DIM), jnp.float32)
                for e in range(_NDIR):
                    pe = jnp.dot(x2, w_ref[e],
                                 preferred_element_type=jnp.float32)
                    pe = pe.reshape(co, _B, _DIM) + bb_ref[e]
                    me = (dm == e).astype(jnp.float32)
                    acc = acc + me * pe
                acc = _prelu(acc, alphas_ref[alpha_idx])
                dst[pl.ds(i * co, co)] = acc
                return 0

            jax.lax.fori_loop(0, n2 // co, up_chunk, 0)

    final = bufs[len(meta) % 2]
    out_ref[...] = final[0]


def kernel(points, vec, dmap, drev, leaf_W, leaf_b, merge_Ws, merge_bs, alphas):
    dims, meta = _layer_meta()

    pts = jnp.transpose(points, (1, 0, 2))
    pts = jnp.pad(pts, ((0, 0), (0, 0), (0, 5)))
    pts = pts.reshape(_N // 16, 16, _B, 8)
    pts = jnp.transpose(pts, (0, 2, 1, 3)).reshape(_N // 16, _B, _DIM)

    lWp = jnp.pad(leaf_W, ((0, 5), (0, 0)))
    leafW = jnp.zeros((_DIM, _DIM), jnp.float32)
    for k in range(16):
        leafW = jax.lax.dynamic_update_slice(leafW, lWp, (k * 8, k * 8))
    leafb = jnp.tile(leaf_b, 16).reshape(1, _DIM)

    ve_in = []
    w_in = []
    b_in = []
    for m in meta:
        l = m["l"]
        n2, d2 = m["n2"], m["d2"]
        v = jax.lax.dynamic_slice(vec, (m["off"],), (n2,))
        ve = jnp.broadcast_to(v.reshape(n2, 1), (n2, d2))
        ve_in.append(ve.reshape(n2 * d2 // _DIM, 1, _DIM))
        W = merge_Ws[l]
        bb = merge_bs[l]
        if m["packed"]:
            g2 = _DIM // d2
            bd = jnp.zeros((_NDIR, _DIM, _DIM), jnp.float32)
            for k in range(g2):
                bd = jax.lax.dynamic_update_slice(bd, W, (0, k * d2, k * d2))
            w_in.append(bd)
            b_in.append(jnp.tile(bb, (1, g2)).reshape(_NDIR, 1, _DIM))
        else:
            w_in.append(W)
            b_in.append(bb.reshape(_NDIR, 1, _DIM))

    smem = pl.BlockSpec(memory_space=pltpu.SMEM)
    vmem = pl.BlockSpec(memory_space=pltpu.VMEM)
    nl = len(meta)
    body = functools.partial(_encoder_body, meta)
    return pl.pallas_call(
        body,
        out_shape=jax.ShapeDtypeStruct((_B, _DIM), jnp.float32),
        in_specs=[vmem, smem, smem, smem, vmem, vmem] + [vmem] * (3 * nl),
        out_specs=vmem,
        scratch_shapes=[
            pltpu.VMEM((1024, _B, _DIM), jnp.float32),
            pltpu.VMEM((1024, _B, _DIM), jnp.float32),
        ],
        compiler_params=pltpu.CompilerParams(
            vmem_limit_bytes=100 * 1024 * 1024),
    )(pts, dmap, drev.astype(jnp.float32), alphas, leafW, leafb,
      *ve_in, *w_in, *b_in)
